# baseline (speedup 1.0000x reference)
import functools

import jax
import jax.numpy as jnp
from jax import lax
from jax.experimental import pallas as pl
from jax.experimental.pallas import tpu as pltpu
from jax.experimental.pallas import tpu_sc as plsc

_NC = 2
_NS = 16
_NW = _NC * _NS
_K = 128


def _lin_relu_body(x_ref, wt_ref, b_ref, o_ref):
    o_ref[...] = jnp.maximum(
        jnp.dot(x_ref[...], wt_ref[...], preferred_element_type=jnp.float32)
        + b_ref[...],
        0.0,
    )


def _lin_relu(x, wt, b2, br):
    n, d = x.shape
    grid = (n // br,)
    return pl.pallas_call(
        _lin_relu_body,
        grid=grid,
        in_specs=[
            pl.BlockSpec((br, d), lambda i: (i, 0)),
            pl.BlockSpec((d, d), lambda i: (0, 0)),
            pl.BlockSpec((1, d), lambda i: (0, 0)),
        ],
        out_specs=pl.BlockSpec((br, d), lambda i: (i, 0)),
        out_shape=jax.ShapeDtypeStruct((n, d), jnp.float32),
    )(x, wt, b2)


def _sc_aggregate(m, src, dst, n, d, n_pad, e):
    zr = n_pad // _NS
    total_chunks = e // _K
    q = total_chunks // _NW
    r = total_chunks - q * _NW
    if q % 2:
        q -= 1
        r += _NW
    mesh = plsc.VectorSubcoreMesh(core_axis_name="c", subcore_axis_name="s")

    @functools.partial(
        pl.kernel,
        out_type=jax.ShapeDtypeStruct((_NC, n_pad, d), jnp.float32),
        mesh=mesh,
        scratch_types=[
            pltpu.VMEM((q * _K,), jnp.int32),
            pltpu.VMEM((_K,), jnp.int32),
            pltpu.VMEM((_K,), jnp.int32),
            pltpu.VMEM((_K, d), jnp.float32),
            pltpu.VMEM((_K, d), jnp.float32),
            pltpu.VMEM_SHARED((n_pad, d), jnp.float32),
            pltpu.SemaphoreType.DMA,
            pltpu.SemaphoreType.DMA,
            pltpu.SemaphoreType.DMA,
            pltpu.SemaphoreType.DMA,
            pltpu.SemaphoreType.DMA,
            pltpu.SemaphoreType.DMA,
        ],
    )
    def k(m_hbm, src_hbm, dst_hbm, out_hbm,
          srcb, dv0, dv1, rows0, rows1, acc, g0, g1, d0, d1, s0, s1):
        rows = (rows0, rows1)
        dstv = (dv0, dv1)
        gsem = (g0, g1)
        dsem = (d0, d1)
        ssem = (s0, s1)
        c = lax.axis_index("c")
        s = lax.axis_index("s")
        wid = s * _NC + c
        base = wid * q * _K
        pltpu.sync_copy(src_hbm.at[pl.ds(base, q * _K)], srcb)
        z16 = jnp.zeros((16,), jnp.float32)

        def zrow(i, carry):
            for j in range(d // 16):
                rows0[i, pl.ds(j * 16, 16)] = z16
            return carry

        lax.fori_loop(0, _K, zrow, 0)
        for tt in range(zr // _K):
            pltpu.sync_copy(rows0, acc.at[pl.ds(s * zr + tt * _K, _K)])
        if zr % _K:
            pltpu.sync_copy(
                rows0.at[pl.ds(0, zr % _K)],
                acc.at[pl.ds(s * zr + (zr // _K) * _K, zr % _K)])
        plsc.subcore_barrier()

        def start_fetch(i, b):
            pltpu.async_copy(m_hbm.at[srcb.at[pl.ds(i * _K, _K)]], rows[b],
                             gsem[b])
            pltpu.async_copy(dst_hbm.at[pl.ds(base + i * _K, _K)],
                             dstv[b], dsem[b])

        def wait_fetch(b):
            pltpu.make_async_copy(m_hbm.at[srcb.at[pl.ds(0, _K)]], rows[b],
                                  gsem[b]).wait()
            pltpu.make_async_copy(dst_hbm.at[pl.ds(base, _K)], dstv[b],
                                  dsem[b]).wait()

        def start_scatter(b):
            pltpu.async_copy(rows[b], acc.at[dstv[b]], ssem[b], add=True)

        def wait_scatter(b):
            pltpu.make_async_copy(rows[b], acc.at[dstv[b]], ssem[b]).wait()

        start_fetch(0, 0)
        start_fetch(1, 1)
        wait_fetch(0)
        start_scatter(0)

        def pair(j, carry):
            for b, i_off in ((1, 1), (0, 2)):
                i = 2 * j + i_off
                nb = 1 - b
                wait_scatter(nb)
                start_fetch(i + 1, nb)
                wait_fetch(b)
                start_scatter(b)
            return carry

        lax.fori_loop(0, (q - 2) // 2, pair, 0)
        wait_scatter(0)
        wait_fetch(1)
        start_scatter(1)
        wait_scatter(1)

        for rnd in range((r + _NW - 1) // _NW):
            xc = rnd * _NW + wid

            @pl.when(xc < r)
            def _():
                xbase = (_NW * q + xc) * _K
                pltpu.sync_copy(src_hbm.at[pl.ds(xbase, _K)],
                                srcb.at[pl.ds(0, _K)])
                pltpu.async_copy(dst_hbm.at[pl.ds(xbase, _K)], dv0, d0)
                pltpu.async_copy(m_hbm.at[srcb.at[pl.ds(0, _K)]], rows0, g0)
                pltpu.make_async_copy(dst_hbm.at[pl.ds(xbase, _K)], dv0,
                                      d0).wait()
                pltpu.make_async_copy(m_hbm.at[srcb.at[pl.ds(0, _K)]], rows0,
                                      g0).wait()
                pltpu.sync_copy(rows0, acc.at[dv0], add=True)

        plsc.subcore_barrier()
        pltpu.sync_copy(
            acc.at[pl.ds(s * zr, zr)],
            out_hbm.at[c, pl.ds(s * zr, zr), :],
        )

    return k(m, src, dst)


def _mlp_body(x_ref, p0_ref, p1_ref, eps_ref, w1t_ref, g_ref, bt_ref,
              w2t_ref, inv_n_ref, o_ref, h1_scr, st_scr):
    ph = pl.program_id(0)
    i = pl.program_id(1)
    br = x_ref.shape[0]

    @pl.when(ph == 0)
    def _():
        h = x_ref[...] * (1.0 + eps_ref[0, 0]) + p0_ref[0] + p1_ref[0]
        h1 = jnp.dot(h, w1t_ref[...], preferred_element_type=jnp.float32)
        h1_scr[pl.ds(i * br, br), :] = h1
        s = jnp.sum(h1, axis=0, keepdims=True)
        s2 = jnp.sum(h1 * h1, axis=0, keepdims=True)
        blk = jnp.concatenate(
            [s, s2, jnp.zeros((6, s.shape[1]), jnp.float32)], axis=0)

        @pl.when(i == 0)
        def _():
            st_scr[...] = jnp.zeros_like(st_scr)

        st_scr[...] += blk

    @pl.when(ph == 1)
    def _():
        inv_n = inv_n_ref[0, 0]
        st = st_scr[...]
        mean = st[0:1, :] * inv_n
        var = st[1:2, :] * inv_n - mean * mean
        inv = lax.rsqrt(var + 1e-5)
        h1 = h1_scr[pl.ds(i * br, br), :]
        h1n = (h1 - mean) * (inv * g_ref[...]) + bt_ref[...]
        o_ref[...] = jnp.dot(
            jnp.maximum(h1n, 0.0), w2t_ref[...],
            preferred_element_type=jnp.float32)


def _mlp(x, partials, eps2, w1t, g2, bt2, w2t, inv_n, br):
    n, d = x.shape
    grid = (2, n // br)
    row = pl.BlockSpec((br, d), lambda p, i: (i, 0))
    prow0 = pl.BlockSpec((1, br, d), lambda p, i: (0, i, 0))
    prow1 = pl.BlockSpec((1, br, d), lambda p, i: (1, i, 0))
    full = lambda shape: pl.BlockSpec(shape, lambda p, i: (0, 0))
    return pl.pallas_call(
        _mlp_body,
        grid=grid,
        in_specs=[
            row,
            prow0,
            prow1,
            full((1, 1)),
            full((d, d)),
            full((1, d)),
            full((1, d)),
            full((d, d)),
            full((1, 1)),
        ],
        out_specs=row,
        out_shape=jax.ShapeDtypeStruct((n, d), jnp.float32),
        scratch_shapes=[
            pltpu.VMEM((n, d), jnp.float32),
            pltpu.VMEM((8, d), jnp.float32),
        ],
    )(x, partials, partials, eps2, w1t, g2, bt2, w2t, inv_n)


def kernel(x, edge_index, W_lin, b_lin, eps, W1, gamma, beta, W2):
    n, d = x.shape
    e = edge_index.shape[1]
    br = 2000
    assert e % _K == 0 and e // _K >= 2 * _NW
    n_pad = -(-n // 128) * 128

    dst = edge_index[0]
    src = edge_index[1]

    m = _lin_relu(x, W_lin.T, b_lin.reshape(1, d), br)
    partials = _sc_aggregate(m, src, dst, n, d, n_pad, e)

    inv_n = jnp.full((1, 1), 1.0 / n, jnp.float32)
    out = _mlp(x, partials, eps.reshape(1, 1), W1.T,
               gamma.reshape(1, d), beta.reshape(1, d), W2.T, inv_n, br)
    return out


# speedup vs baseline: 1.0804x; 1.0804x over previous
import functools

import jax
import jax.numpy as jnp
from jax import lax
from jax.experimental import pallas as pl
from jax.experimental.pallas import tpu as pltpu
from jax.experimental.pallas import tpu_sc as plsc

_NC = 2
_NS = 16
_NW = _NC * _NS
_K = 128


def _lin_relu_body(x_ref, wt_ref, b_ref, o_ref):
    o_ref[...] = jnp.maximum(
        jnp.dot(x_ref[...], wt_ref[...], preferred_element_type=jnp.float32)
        + b_ref[...],
        0.0,
    )


def _lin_relu(x, wt, b2, br):
    n, d = x.shape
    grid = (n // br,)
    return pl.pallas_call(
        _lin_relu_body,
        grid=grid,
        in_specs=[
            pl.BlockSpec((br, d), lambda i: (i, 0)),
            pl.BlockSpec((d, d), lambda i: (0, 0)),
            pl.BlockSpec((1, d), lambda i: (0, 0)),
        ],
        out_specs=pl.BlockSpec((br, d), lambda i: (i, 0)),
        out_shape=jax.ShapeDtypeStruct((n, d), jnp.float32),
    )(x, wt, b2)


def _sc_aggregate(m, src1, ei, n, d, n_pad, e):
    zr = n_pad // _NS
    total_chunks = e // _K
    q = total_chunks // _NW
    r = total_chunks - q * _NW
    if q % 2:
        q -= 1
        r += _NW
    mesh = plsc.VectorSubcoreMesh(core_axis_name="c", subcore_axis_name="s")

    @functools.partial(
        pl.kernel,
        out_type=jax.ShapeDtypeStruct((_NC, n_pad, d), jnp.float32),
        mesh=mesh,
        scratch_types=[
            pltpu.VMEM((q * _K,), jnp.int32),
            pltpu.VMEM((_K,), jnp.int32),
            pltpu.VMEM((_K,), jnp.int32),
            pltpu.VMEM((_K, d), jnp.float32),
            pltpu.VMEM((_K, d), jnp.float32),
            pltpu.VMEM_SHARED((n_pad, d), jnp.float32),
            pltpu.SemaphoreType.DMA,
            pltpu.SemaphoreType.DMA,
            pltpu.SemaphoreType.DMA,
            pltpu.SemaphoreType.DMA,
            pltpu.SemaphoreType.DMA,
            pltpu.SemaphoreType.DMA,
        ],
    )
    def k(m_hbm, src_hbm, ei_hbm, out_hbm,
          srcb, dv0, dv1, rows0, rows1, acc, g0, g1, d0, d1, s0, s1):
        rows = (rows0, rows1)
        dstv = (dv0, dv1)
        gsem = (g0, g1)
        dsem = (d0, d1)
        ssem = (s0, s1)
        c = lax.axis_index("c")
        s = lax.axis_index("s")
        wid = s * _NC + c
        base = wid * q * _K
        pltpu.sync_copy(src_hbm.at[0, pl.ds(base, q * _K)], srcb)
        z16 = jnp.zeros((16,), jnp.float32)

        def zrow(i, carry):
            for j in range(d // 16):
                rows0[i, pl.ds(j * 16, 16)] = z16
            return carry

        lax.fori_loop(0, _K, zrow, 0)
        for tt in range(zr // _K):
            pltpu.sync_copy(rows0, acc.at[pl.ds(s * zr + tt * _K, _K)])
        if zr % _K:
            pltpu.sync_copy(
                rows0.at[pl.ds(0, zr % _K)],
                acc.at[pl.ds(s * zr + (zr // _K) * _K, zr % _K)])
        plsc.subcore_barrier()

        def start_fetch(i, b):
            pltpu.async_copy(m_hbm.at[srcb.at[pl.ds(i * _K, _K)]], rows[b],
                             gsem[b])
            pltpu.async_copy(ei_hbm.at[0, pl.ds(base + i * _K, _K)],
                             dstv[b], dsem[b])

        def wait_fetch(b):
            pltpu.make_async_copy(m_hbm.at[srcb.at[pl.ds(0, _K)]], rows[b],
                                  gsem[b]).wait()
            pltpu.make_async_copy(ei_hbm.at[0, pl.ds(base, _K)], dstv[b],
                                  dsem[b]).wait()

        def start_scatter(b):
            pltpu.async_copy(rows[b], acc.at[dstv[b]], ssem[b], add=True)

        def wait_scatter(b):
            pltpu.make_async_copy(rows[b], acc.at[dstv[b]], ssem[b]).wait()

        start_fetch(0, 0)
        start_fetch(1, 1)
        wait_fetch(0)
        start_scatter(0)

        def pair(j, carry):
            for b, i_off in ((1, 1), (0, 2)):
                i = 2 * j + i_off
                nb = 1 - b
                wait_scatter(nb)
                start_fetch(i + 1, nb)
                wait_fetch(b)
                start_scatter(b)
            return carry

        lax.fori_loop(0, (q - 2) // 2, pair, 0)
        wait_scatter(0)
        wait_fetch(1)
        start_scatter(1)
        wait_scatter(1)

        for rnd in range((r + _NW - 1) // _NW):
            xc = rnd * _NW + wid

            @pl.when(xc < r)
            def _():
                xbase = (_NW * q + xc) * _K
                pltpu.sync_copy(src_hbm.at[0, pl.ds(xbase, _K)],
                                srcb.at[pl.ds(0, _K)])
                pltpu.async_copy(ei_hbm.at[0, pl.ds(xbase, _K)], dv0, d0)
                pltpu.async_copy(m_hbm.at[srcb.at[pl.ds(0, _K)]], rows0, g0)
                pltpu.make_async_copy(ei_hbm.at[0, pl.ds(xbase, _K)], dv0,
                                      d0).wait()
                pltpu.make_async_copy(m_hbm.at[srcb.at[pl.ds(0, _K)]], rows0,
                                      g0).wait()
                pltpu.sync_copy(rows0, acc.at[dv0], add=True)

        plsc.subcore_barrier()
        pltpu.sync_copy(
            acc.at[pl.ds(s * zr, zr)],
            out_hbm.at[c, pl.ds(s * zr, zr), :],
        )

    return k(m, src1, ei)


def _mlp_body(x_ref, p0_ref, p1_ref, eps_ref, w1t_ref, g_ref, bt_ref,
              w2t_ref, inv_n_ref, o_ref, h1_scr, st_scr):
    ph = pl.program_id(0)
    i = pl.program_id(1)
    br = x_ref.shape[0]

    @pl.when(ph == 0)
    def _():
        h = x_ref[...] * (1.0 + eps_ref[0, 0]) + p0_ref[0] + p1_ref[0]
        h1 = jnp.dot(h, w1t_ref[...], preferred_element_type=jnp.float32)
        h1_scr[pl.ds(i * br, br), :] = h1
        s = jnp.sum(h1, axis=0, keepdims=True)
        s2 = jnp.sum(h1 * h1, axis=0, keepdims=True)
        blk = jnp.concatenate(
            [s, s2, jnp.zeros((6, s.shape[1]), jnp.float32)], axis=0)

        @pl.when(i == 0)
        def _():
            st_scr[...] = jnp.zeros_like(st_scr)

        st_scr[...] += blk

    @pl.when(ph == 1)
    def _():
        inv_n = inv_n_ref[0, 0]
        st = st_scr[...]
        mean = st[0:1, :] * inv_n
        var = st[1:2, :] * inv_n - mean * mean
        inv = lax.rsqrt(var + 1e-5)
        h1 = h1_scr[pl.ds(i * br, br), :]
        h1n = (h1 - mean) * (inv * g_ref[...]) + bt_ref[...]
        o_ref[...] = jnp.dot(
            jnp.maximum(h1n, 0.0), w2t_ref[...],
            preferred_element_type=jnp.float32)


def _mlp(x, partials, eps2, w1t, g2, bt2, w2t, inv_n, br):
    n, d = x.shape
    grid = (2, n // br)
    row = pl.BlockSpec((br, d), lambda p, i: (i, 0))
    prow0 = pl.BlockSpec((1, br, d), lambda p, i: (0, i, 0))
    prow1 = pl.BlockSpec((1, br, d), lambda p, i: (1, i, 0))
    full = lambda shape: pl.BlockSpec(shape, lambda p, i: (0, 0))
    return pl.pallas_call(
        _mlp_body,
        grid=grid,
        in_specs=[
            row,
            prow0,
            prow1,
            full((1, 1)),
            full((d, d)),
            full((1, d)),
            full((1, d)),
            full((d, d)),
            full((1, 1)),
        ],
        out_specs=row,
        out_shape=jax.ShapeDtypeStruct((n, d), jnp.float32),
        scratch_shapes=[
            pltpu.VMEM((n, d), jnp.float32),
            pltpu.VMEM((8, d), jnp.float32),
        ],
    )(x, partials, partials, eps2, w1t, g2, bt2, w2t, inv_n)


def kernel(x, edge_index, W_lin, b_lin, eps, W1, gamma, beta, W2):
    n, d = x.shape
    e = edge_index.shape[1]
    br = 2000
    assert e % _K == 0 and e // _K >= 2 * _NW
    n_pad = -(-n // 128) * 128

    src1 = lax.slice(edge_index, (1, 0), (2, e))

    m = _lin_relu(x, W_lin.T, b_lin.reshape(1, d), br)
    partials = _sc_aggregate(m, src1, edge_index, n, d, n_pad, e)

    inv_n = jnp.full((1, 1), 1.0 / n, jnp.float32)
    out = _mlp(x, partials, eps.reshape(1, 1), W1.T,
               gamma.reshape(1, d), beta.reshape(1, d), W2.T, inv_n, br)
    return out


# speedup vs baseline: 1.1141x; 1.0312x over previous
import functools

import jax
import jax.numpy as jnp
from jax import lax
from jax.experimental import pallas as pl
from jax.experimental.pallas import tpu as pltpu
from jax.experimental.pallas import tpu_sc as plsc

_NC = 2
_NS = 16
_NW = _NC * _NS
_K = 128


def _lin_relu_body(x_ref, wt_ref, b_ref, o_ref):
    o_ref[...] = jnp.maximum(
        jnp.dot(x_ref[...], wt_ref[...], preferred_element_type=jnp.float32)
        + b_ref[...],
        0.0,
    )


def _lin_relu(x, wt, b2, br):
    n, d = x.shape
    grid = (n // br,)
    return pl.pallas_call(
        _lin_relu_body,
        grid=grid,
        in_specs=[
            pl.BlockSpec((br, d), lambda i: (i, 0)),
            pl.BlockSpec((d, d), lambda i: (0, 0)),
            pl.BlockSpec((1, d), lambda i: (0, 0)),
        ],
        out_specs=pl.BlockSpec((br, d), lambda i: (i, 0)),
        out_shape=jax.ShapeDtypeStruct((n, d), jnp.float32),
    )(x, wt, b2)


def _sc_aggregate(m, ei, n, d, n_pad, e):
    zr = n_pad // _NS
    total_chunks = e // _K
    q = total_chunks // _NW
    r = total_chunks - q * _NW
    if q % 2:
        q -= 1
        r += _NW
    mesh = plsc.VectorSubcoreMesh(core_axis_name="c", subcore_axis_name="s")

    @functools.partial(
        pl.kernel,
        out_type=jax.ShapeDtypeStruct((_NC, n_pad, d), jnp.float32),
        mesh=mesh,
        scratch_types=[
            pltpu.VMEM((q * _K,), jnp.int32),
            pltpu.VMEM((_K,), jnp.int32),
            pltpu.VMEM((_K,), jnp.int32),
            pltpu.VMEM((_K, d), jnp.float32),
            pltpu.VMEM((_K, d), jnp.float32),
            pltpu.VMEM_SHARED((n_pad, d), jnp.float32),
            pltpu.SemaphoreType.DMA,
            pltpu.SemaphoreType.DMA,
            pltpu.SemaphoreType.DMA,
            pltpu.SemaphoreType.DMA,
            pltpu.SemaphoreType.DMA,
            pltpu.SemaphoreType.DMA,
        ],
    )
    def k(m_hbm, ei_hbm, out_hbm,
          srcb, dv0, dv1, rows0, rows1, acc, g0, g1, d0, d1, s0, s1):
        rows = (rows0, rows1)
        dstv = (dv0, dv1)
        gsem = (g0, g1)
        dsem = (d0, d1)
        ssem = (s0, s1)
        c = lax.axis_index("c")
        s = lax.axis_index("s")
        wid = s * _NC + c
        base = wid * q * _K
        pltpu.sync_copy(ei_hbm.at[1, pl.ds(base, q * _K)], srcb)
        z16 = jnp.zeros((16,), jnp.float32)

        def zrow(i, carry):
            for j in range(d // 16):
                rows0[i, pl.ds(j * 16, 16)] = z16
            return carry

        lax.fori_loop(0, _K, zrow, 0)
        for tt in range(zr // _K):
            pltpu.sync_copy(rows0, acc.at[pl.ds(s * zr + tt * _K, _K)])
        if zr % _K:
            pltpu.sync_copy(
                rows0.at[pl.ds(0, zr % _K)],
                acc.at[pl.ds(s * zr + (zr // _K) * _K, zr % _K)])
        plsc.subcore_barrier()

        def start_fetch(i, b):
            pltpu.async_copy(m_hbm.at[srcb.at[pl.ds(i * _K, _K)]], rows[b],
                             gsem[b])
            pltpu.async_copy(ei_hbm.at[0, pl.ds(base + i * _K, _K)],
                             dstv[b], dsem[b])

        def wait_fetch(b):
            pltpu.make_async_copy(m_hbm.at[srcb.at[pl.ds(0, _K)]], rows[b],
                                  gsem[b]).wait()
            pltpu.make_async_copy(ei_hbm.at[0, pl.ds(base, _K)], dstv[b],
                                  dsem[b]).wait()

        def start_scatter(b):
            pltpu.async_copy(rows[b], acc.at[dstv[b]], ssem[b], add=True)

        def wait_scatter(b):
            pltpu.make_async_copy(rows[b], acc.at[dstv[b]], ssem[b]).wait()

        start_fetch(0, 0)
        start_fetch(1, 1)
        wait_fetch(0)
        start_scatter(0)

        def pair(j, carry):
            for b, i_off in ((1, 1), (0, 2)):
                i = 2 * j + i_off
                nb = 1 - b
                wait_scatter(nb)
                start_fetch(i + 1, nb)
                wait_fetch(b)
                start_scatter(b)
            return carry

        lax.fori_loop(0, (q - 2) // 2, pair, 0)
        wait_scatter(0)
        wait_fetch(1)
        start_scatter(1)
        wait_scatter(1)

        for rnd in range((r + _NW - 1) // _NW):
            xc = rnd * _NW + wid

            @pl.when(xc < r)
            def _():
                xbase = (_NW * q + xc) * _K
                pltpu.sync_copy(ei_hbm.at[1, pl.ds(xbase, _K)],
                                srcb.at[pl.ds(0, _K)])
                pltpu.async_copy(ei_hbm.at[0, pl.ds(xbase, _K)], dv0, d0)
                pltpu.async_copy(m_hbm.at[srcb.at[pl.ds(0, _K)]], rows0, g0)
                pltpu.make_async_copy(ei_hbm.at[0, pl.ds(xbase, _K)], dv0,
                                      d0).wait()
                pltpu.make_async_copy(m_hbm.at[srcb.at[pl.ds(0, _K)]], rows0,
                                      g0).wait()
                pltpu.sync_copy(rows0, acc.at[dv0], add=True)

        plsc.subcore_barrier()
        pltpu.sync_copy(
            acc.at[pl.ds(s * zr, zr)],
            out_hbm.at[c, pl.ds(s * zr, zr), :],
        )

    return k(m, ei)


def _mlp_body(x_ref, p0_ref, p1_ref, eps_ref, w1t_ref, g_ref, bt_ref,
              w2t_ref, inv_n_ref, o_ref, h1_scr, st_scr):
    ph = pl.program_id(0)
    i = pl.program_id(1)
    br = x_ref.shape[0]

    @pl.when(ph == 0)
    def _():
        h = x_ref[...] * (1.0 + eps_ref[0, 0]) + p0_ref[0] + p1_ref[0]
        h1 = jnp.dot(h, w1t_ref[...], preferred_element_type=jnp.float32)
        h1_scr[pl.ds(i * br, br), :] = h1
        s = jnp.sum(h1, axis=0, keepdims=True)
        s2 = jnp.sum(h1 * h1, axis=0, keepdims=True)
        blk = jnp.concatenate(
            [s, s2, jnp.zeros((6, s.shape[1]), jnp.float32)], axis=0)

        @pl.when(i == 0)
        def _():
            st_scr[...] = jnp.zeros_like(st_scr)

        st_scr[...] += blk

    @pl.when(ph == 1)
    def _():
        inv_n = inv_n_ref[0, 0]
        st = st_scr[...]
        mean = st[0:1, :] * inv_n
        var = st[1:2, :] * inv_n - mean * mean
        inv = lax.rsqrt(var + 1e-5)
        h1 = h1_scr[pl.ds(i * br, br), :]
        h1n = (h1 - mean) * (inv * g_ref[...]) + bt_ref[...]
        o_ref[...] = jnp.dot(
            jnp.maximum(h1n, 0.0), w2t_ref[...],
            preferred_element_type=jnp.float32)


def _mlp(x, partials, eps2, w1t, g2, bt2, w2t, inv_n, br):
    n, d = x.shape
    grid = (2, n // br)
    row = pl.BlockSpec((br, d), lambda p, i: (i, 0))
    prow0 = pl.BlockSpec((1, br, d), lambda p, i: (0, i, 0))
    prow1 = pl.BlockSpec((1, br, d), lambda p, i: (1, i, 0))
    full = lambda shape: pl.BlockSpec(shape, lambda p, i: (0, 0))
    return pl.pallas_call(
        _mlp_body,
        grid=grid,
        in_specs=[
            row,
            prow0,
            prow1,
            full((1, 1)),
            full((d, d)),
            full((1, d)),
            full((1, d)),
            full((d, d)),
            full((1, 1)),
        ],
        out_specs=row,
        out_shape=jax.ShapeDtypeStruct((n, d), jnp.float32),
        scratch_shapes=[
            pltpu.VMEM((n, d), jnp.float32),
            pltpu.VMEM((8, d), jnp.float32),
        ],
    )(x, partials, partials, eps2, w1t, g2, bt2, w2t, inv_n)


def kernel(x, edge_index, W_lin, b_lin, eps, W1, gamma, beta, W2):
    n, d = x.shape
    e = edge_index.shape[1]
    br = 2000
    assert e % _K == 0 and e // _K >= 2 * _NW
    n_pad = -(-n // 128) * 128

    m = _lin_relu(x, W_lin.T, b_lin.reshape(1, d), br)
    partials = _sc_aggregate(m, edge_index, n, d, n_pad, e)

    inv_n = jnp.full((1, 1), 1.0 / n, jnp.float32)
    out = _mlp(x, partials, eps.reshape(1, 1), W1.T,
               gamma.reshape(1, d), beta.reshape(1, d), W2.T, inv_n, br)
    return out


# speedup vs baseline: 1.1297x; 1.0140x over previous
import functools

import jax
import jax.numpy as jnp
from jax import lax
from jax.experimental import pallas as pl
from jax.experimental.pallas import tpu as pltpu
from jax.experimental.pallas import tpu_sc as plsc

_NC = 2
_NS = 16
_NW = _NC * _NS
_K = 128


def _lin_relu_body(x_ref, wt_ref, b_ref, o_ref):
    o_ref[...] = jnp.maximum(
        jnp.dot(x_ref[...], wt_ref[...], preferred_element_type=jnp.float32)
        + b_ref[...],
        0.0,
    )


def _lin_relu(x, wt, b2, br):
    n, d = x.shape
    grid = (n // br,)
    return pl.pallas_call(
        _lin_relu_body,
        grid=grid,
        in_specs=[
            pl.BlockSpec((br, d), lambda i: (i, 0)),
            pl.BlockSpec((d, d), lambda i: (0, 0)),
            pl.BlockSpec((1, d), lambda i: (0, 0)),
        ],
        out_specs=pl.BlockSpec((br, d), lambda i: (i, 0)),
        out_shape=jax.ShapeDtypeStruct((n, d), jnp.float32),
    )(x, wt, b2)


def _sc_aggregate(m, ei, n, d, n_pad, e):
    zr = n_pad // _NS
    total_chunks = e // _K
    q = total_chunks // _NW
    r = total_chunks - q * _NW
    if q % 2:
        q -= 1
        r += _NW
    mesh = plsc.VectorSubcoreMesh(core_axis_name="c", subcore_axis_name="s")

    @functools.partial(
        pl.kernel,
        out_type=jax.ShapeDtypeStruct((_NC, n_pad, d), jnp.float32),
        mesh=mesh,
        scratch_types=[
            pltpu.VMEM((q * _K,), jnp.int32),
            pltpu.VMEM((_K,), jnp.int32),
            pltpu.VMEM((_K,), jnp.int32),
            pltpu.VMEM((_K, d), jnp.float32),
            pltpu.VMEM((_K, d), jnp.float32),
            pltpu.VMEM_SHARED((n_pad, d), jnp.float32),
            pltpu.SemaphoreType.DMA,
            pltpu.SemaphoreType.DMA,
            pltpu.SemaphoreType.DMA,
            pltpu.SemaphoreType.DMA,
            pltpu.SemaphoreType.DMA,
            pltpu.SemaphoreType.DMA,
        ],
    )
    def k(m_hbm, ei_hbm, out_hbm,
          srcb, dv0, dv1, rows0, rows1, acc, g0, g1, d0, d1, s0, s1):
        rows = (rows0, rows1)
        dstv = (dv0, dv1)
        gsem = (g0, g1)
        dsem = (d0, d1)
        ssem = (s0, s1)
        c = lax.axis_index("c")
        s = lax.axis_index("s")
        wid = s * _NC + c
        base = wid * q * _K
        pltpu.async_copy(ei_hbm.at[1, pl.ds(base, q * _K)], srcb, g0)
        z16 = jnp.zeros((16,), jnp.float32)

        def zrow(i, carry):
            for j in range(d // 16):
                rows1[i, pl.ds(j * 16, 16)] = z16
            return carry

        lax.fori_loop(0, _K, zrow, 0)
        pltpu.make_async_copy(ei_hbm.at[1, pl.ds(base, q * _K)], srcb,
                              g0).wait()

        def start_fetch(i, b):
            pltpu.async_copy(m_hbm.at[srcb.at[pl.ds(i * _K, _K)]], rows[b],
                             gsem[b])
            pltpu.async_copy(ei_hbm.at[0, pl.ds(base + i * _K, _K)],
                             dstv[b], dsem[b])

        def wait_fetch(b):
            pltpu.make_async_copy(m_hbm.at[srcb.at[pl.ds(0, _K)]], rows[b],
                                  gsem[b]).wait()
            pltpu.make_async_copy(ei_hbm.at[0, pl.ds(base, _K)], dstv[b],
                                  dsem[b]).wait()

        def start_scatter(b):
            pltpu.async_copy(rows[b], acc.at[dstv[b]], ssem[b], add=True)

        def wait_scatter(b):
            pltpu.make_async_copy(rows[b], acc.at[dstv[b]], ssem[b]).wait()

        start_fetch(0, 0)
        for tt in range(zr // _K):
            pltpu.sync_copy(rows1, acc.at[pl.ds(s * zr + tt * _K, _K)])
        if zr % _K:
            pltpu.sync_copy(
                rows1.at[pl.ds(0, zr % _K)],
                acc.at[pl.ds(s * zr + (zr // _K) * _K, zr % _K)])
        plsc.subcore_barrier()
        start_fetch(1, 1)
        wait_fetch(0)
        start_scatter(0)

        def pair(j, carry):
            for b, i_off in ((1, 1), (0, 2)):
                i = 2 * j + i_off
                nb = 1 - b
                wait_scatter(nb)
                start_fetch(i + 1, nb)
                wait_fetch(b)
                start_scatter(b)
            return carry

        lax.fori_loop(0, (q - 2) // 2, pair, 0)
        wait_scatter(0)
        wait_fetch(1)
        start_scatter(1)
        wait_scatter(1)

        for rnd in range((r + _NW - 1) // _NW):
            xc = rnd * _NW + wid

            @pl.when(xc < r)
            def _():
                xbase = (_NW * q + xc) * _K
                pltpu.sync_copy(ei_hbm.at[1, pl.ds(xbase, _K)],
                                srcb.at[pl.ds(0, _K)])
                pltpu.async_copy(ei_hbm.at[0, pl.ds(xbase, _K)], dv0, d0)
                pltpu.async_copy(m_hbm.at[srcb.at[pl.ds(0, _K)]], rows0, g0)
                pltpu.make_async_copy(ei_hbm.at[0, pl.ds(xbase, _K)], dv0,
                                      d0).wait()
                pltpu.make_async_copy(m_hbm.at[srcb.at[pl.ds(0, _K)]], rows0,
                                      g0).wait()
                pltpu.sync_copy(rows0, acc.at[dv0], add=True)

        plsc.subcore_barrier()
        pltpu.sync_copy(
            acc.at[pl.ds(s * zr, zr)],
            out_hbm.at[c, pl.ds(s * zr, zr), :],
        )

    return k(m, ei)


def _mlp_body(x_ref, p0_ref, p1_ref, eps_ref, w1t_ref, g_ref, bt_ref,
              w2t_ref, inv_n_ref, o_ref, h1_scr, st_scr):
    ph = pl.program_id(0)
    i = pl.program_id(1)
    br = x_ref.shape[0]

    @pl.when(ph == 0)
    def _():
        h = x_ref[...] * (1.0 + eps_ref[0, 0]) + p0_ref[0] + p1_ref[0]
        h1 = jnp.dot(h, w1t_ref[...], preferred_element_type=jnp.float32)
        h1_scr[pl.ds(i * br, br), :] = h1
        s = jnp.sum(h1, axis=0, keepdims=True)
        s2 = jnp.sum(h1 * h1, axis=0, keepdims=True)
        blk = jnp.concatenate(
            [s, s2, jnp.zeros((6, s.shape[1]), jnp.float32)], axis=0)

        @pl.when(i == 0)
        def _():
            st_scr[...] = jnp.zeros_like(st_scr)

        st_scr[...] += blk

    @pl.when(ph == 1)
    def _():
        inv_n = inv_n_ref[0, 0]
        st = st_scr[...]
        mean = st[0:1, :] * inv_n
        var = st[1:2, :] * inv_n - mean * mean
        inv = lax.rsqrt(var + 1e-5)
        h1 = h1_scr[pl.ds(i * br, br), :]
        h1n = (h1 - mean) * (inv * g_ref[...]) + bt_ref[...]
        o_ref[...] = jnp.dot(
            jnp.maximum(h1n, 0.0), w2t_ref[...],
            preferred_element_type=jnp.float32)


def _mlp(x, partials, eps2, w1t, g2, bt2, w2t, inv_n, br):
    n, d = x.shape
    grid = (2, n // br)
    row = pl.BlockSpec((br, d), lambda p, i: (i, 0))
    prow0 = pl.BlockSpec((1, br, d), lambda p, i: (0, i, 0))
    prow1 = pl.BlockSpec((1, br, d), lambda p, i: (1, i, 0))
    full = lambda shape: pl.BlockSpec(shape, lambda p, i: (0, 0))
    return pl.pallas_call(
        _mlp_body,
        grid=grid,
        in_specs=[
            row,
            prow0,
            prow1,
            full((1, 1)),
            full((d, d)),
            full((1, d)),
            full((1, d)),
            full((d, d)),
            full((1, 1)),
        ],
        out_specs=row,
        out_shape=jax.ShapeDtypeStruct((n, d), jnp.float32),
        scratch_shapes=[
            pltpu.VMEM((n, d), jnp.float32),
            pltpu.VMEM((8, d), jnp.float32),
        ],
    )(x, partials, partials, eps2, w1t, g2, bt2, w2t, inv_n)


def kernel(x, edge_index, W_lin, b_lin, eps, W1, gamma, beta, W2):
    n, d = x.shape
    e = edge_index.shape[1]
    br = 2000
    assert e % _K == 0 and e // _K >= 2 * _NW
    n_pad = -(-n // 128) * 128

    m = _lin_relu(x, W_lin.T, b_lin.reshape(1, d), br)
    partials = _sc_aggregate(m, edge_index, n, d, n_pad, e)

    inv_n = jnp.full((1, 1), 1.0 / n, jnp.float32)
    out = _mlp(x, partials, eps.reshape(1, 1), W1.T,
               gamma.reshape(1, d), beta.reshape(1, d), W2.T, inv_n, br)
    return out
